# Initial kernel scaffold; baseline (speedup 1.0000x reference)
import functools

import jax
import jax.numpy as jnp
from jax import lax
from jax.experimental import pallas as pl
from jax.experimental.pallas import tpu as pltpu
from jax.experimental.pallas import tpu_sc as plsc

N_CAT = 26
CAT_CARD = 100000
EMB_DIM = 16
HIDDEN = 128

NUM_CORES = 2
NUM_SUBCORES = 16
NUM_WORKERS = NUM_CORES * NUM_SUBCORES

CHUNK = 128
PHASE_CHUNKS = 13


def _sc_gather(tab_flat, gidx):
    G = gidx.shape[0]
    n_per_w = G // NUM_WORKERS
    assert n_per_w * NUM_WORKERS == G
    rows_per_phase = PHASE_CHUNKS * CHUNK
    n_phases = n_per_w // rows_per_phase
    assert n_phases * rows_per_phase == n_per_w

    mesh = plsc.VectorSubcoreMesh(core_axis_name="c", subcore_axis_name="s")

    @functools.partial(
        pl.kernel,
        out_type=jax.ShapeDtypeStruct((G, EMB_DIM), jnp.float32),
        mesh=mesh,
        scratch_types=[
            pltpu.VMEM((n_per_w,), jnp.int32),
            pltpu.VMEM((2, rows_per_phase, EMB_DIM), jnp.float32),
            pltpu.SemaphoreType.DMA,
            pltpu.SemaphoreType.DMA,
        ],
    )
    def k(tab_hbm, idx_hbm, out_hbm, idx_v, buf, gsem, osem):
        wid = lax.axis_index("s") * NUM_CORES + lax.axis_index("c")
        base = wid * n_per_w
        pltpu.sync_copy(idx_hbm.at[pl.ds(base, n_per_w)], idx_v)
        out_cps = [None, None]
        for p in range(n_phases):
            s = p & 1
            if out_cps[s] is not None:
                out_cps[s].wait()
            cps = []
            for j in range(PHASE_CHUNKS):
                off = (p * PHASE_CHUNKS + j) * CHUNK
                cps.append(
                    pltpu.async_copy(
                        tab_hbm.at[idx_v.at[pl.ds(off, CHUNK)]],
                        buf.at[s, pl.ds(j * CHUNK, CHUNK)],
                        gsem,
                    )
                )
            for cp in cps:
                cp.wait()
            out_cps[s] = pltpu.async_copy(
                buf.at[s],
                out_hbm.at[pl.ds(base + p * rows_per_phase, rows_per_phase)],
                osem,
            )
        for cp in out_cps:
            if cp is not None:
                cp.wait()

    return k(tab_flat, gidx)


def _tc_matmul(embs, cont, W1, W2, b2):
    R = embs.shape[0]
    BR = 1024
    assert R % BR == 0
    E = embs.shape[1]
    C = cont.shape[1]

    def mm(e_ref, c_ref, w1_ref, w2_ref, b_ref, o_ref):
        acc = jnp.dot(e_ref[...], w1_ref[...], preferred_element_type=jnp.float32)
        acc = acc + jnp.dot(c_ref[...], w2_ref[...], preferred_element_type=jnp.float32)
        o_ref[...] = acc + b_ref[...]

    return pl.pallas_call(
        mm,
        grid=(R // BR,),
        in_specs=[
            pl.BlockSpec((BR, E), lambda i: (i, 0)),
            pl.BlockSpec((BR, C), lambda i: (i, 0)),
            pl.BlockSpec((E, HIDDEN), lambda i: (0, 0)),
            pl.BlockSpec((C, HIDDEN), lambda i: (0, 0)),
            pl.BlockSpec((1, HIDDEN), lambda i: (0, 0)),
        ],
        out_specs=pl.BlockSpec((BR, HIDDEN), lambda i: (i, 0)),
        out_shape=jax.ShapeDtypeStruct((R, HIDDEN), jnp.float32),
    )(embs, cont, W1, W2, b2)


def kernel(x, tables, W, b):
    bs, ms, nr, d = x.shape
    R = bs * ms * nr
    xf = x.reshape(R, d)
    offsets = (jnp.arange(N_CAT, dtype=jnp.int32) * CAT_CARD)[None, :]
    gidx = (xf[:, :N_CAT].astype(jnp.int32) + offsets).reshape(R * N_CAT)
    cont = xf[:, N_CAT:]
    tab_flat = tables.reshape(N_CAT * CAT_CARD, EMB_DIM)
    embs = _sc_gather(tab_flat, gidx).reshape(R, N_CAT * EMB_DIM)
    W1 = W[: N_CAT * EMB_DIM]
    W2 = W[N_CAT * EMB_DIM :]
    out = _tc_matmul(embs, cont, W1, W2, b.reshape(1, HIDDEN))
    return out.reshape(bs, ms, nr, HIDDEN)


# speedup vs baseline: 7.8353x; 7.8353x over previous
import functools

import jax
import jax.numpy as jnp
from jax import lax
from jax.experimental import pallas as pl
from jax.experimental.pallas import tpu as pltpu
from jax.experimental.pallas import tpu_sc as plsc

N_CAT = 26
CAT_CARD = 100000
EMB_DIM = 16
HIDDEN = 128

NUM_CORES = 2
NUM_SUBCORES = 16
NUM_WORKERS = NUM_CORES * NUM_SUBCORES

CHUNK = 128
PHASE_CHUNKS = 13


def _sc_gather(tab_flat, gidx):
    G = gidx.shape[0]
    n_per_w = G // NUM_WORKERS
    assert n_per_w * NUM_WORKERS == G
    rows_per_phase = PHASE_CHUNKS * CHUNK
    n_phases = n_per_w // rows_per_phase
    assert n_phases * rows_per_phase == n_per_w

    mesh = plsc.VectorSubcoreMesh(core_axis_name="c", subcore_axis_name="s")

    @functools.partial(
        pl.kernel,
        out_type=jax.ShapeDtypeStruct((G, EMB_DIM), jnp.float32),
        mesh=mesh,
        compiler_params=pltpu.CompilerParams(use_tc_tiling_on_sc=False),
        scratch_types=[
            pltpu.VMEM((n_per_w,), jnp.int32),
            pltpu.VMEM((2, rows_per_phase, EMB_DIM), jnp.float32),
            pltpu.SemaphoreType.DMA,
            pltpu.SemaphoreType.DMA,
        ],
    )
    def k(tab_hbm, idx_hbm, out_hbm, idx_v, buf, gsem, osem):
        wid = lax.axis_index("s") * NUM_CORES + lax.axis_index("c")
        base = wid * n_per_w
        pltpu.sync_copy(idx_hbm.at[pl.ds(base, n_per_w)], idx_v)
        out_cps = [None, None]
        for p in range(n_phases):
            s = p & 1
            if out_cps[s] is not None:
                out_cps[s].wait()
            cps = []
            for j in range(PHASE_CHUNKS):
                off = (p * PHASE_CHUNKS + j) * CHUNK
                cps.append(
                    pltpu.async_copy(
                        tab_hbm.at[idx_v.at[pl.ds(off, CHUNK)]],
                        buf.at[s, pl.ds(j * CHUNK, CHUNK)],
                        gsem,
                    )
                )
            for cp in cps:
                cp.wait()
            out_cps[s] = pltpu.async_copy(
                buf.at[s],
                out_hbm.at[pl.ds(base + p * rows_per_phase, rows_per_phase)],
                osem,
            )
        for cp in out_cps:
            if cp is not None:
                cp.wait()

    return k(tab_flat, gidx)


def _tc_matmul(embs, cont, W1, W2, b2):
    R = embs.shape[0]
    BR = 1024
    assert R % BR == 0
    E = embs.shape[1]
    C = cont.shape[1]

    def mm(e_ref, c_ref, w1_ref, w2_ref, b_ref, o_ref):
        acc = jnp.dot(e_ref[...], w1_ref[...], preferred_element_type=jnp.float32)
        acc = acc + jnp.dot(c_ref[...], w2_ref[...], preferred_element_type=jnp.float32)
        o_ref[...] = acc + b_ref[...]

    return pl.pallas_call(
        mm,
        grid=(R // BR,),
        in_specs=[
            pl.BlockSpec((BR, E), lambda i: (i, 0)),
            pl.BlockSpec((BR, C), lambda i: (i, 0)),
            pl.BlockSpec((E, HIDDEN), lambda i: (0, 0)),
            pl.BlockSpec((C, HIDDEN), lambda i: (0, 0)),
            pl.BlockSpec((1, HIDDEN), lambda i: (0, 0)),
        ],
        out_specs=pl.BlockSpec((BR, HIDDEN), lambda i: (i, 0)),
        out_shape=jax.ShapeDtypeStruct((R, HIDDEN), jnp.float32),
    )(embs, cont, W1, W2, b2)


def kernel(x, tables, W, b):
    bs, ms, nr, d = x.shape
    R = bs * ms * nr
    xf = x.reshape(R, d)
    offsets = (jnp.arange(N_CAT, dtype=jnp.int32) * CAT_CARD)[None, :]
    gidx = (xf[:, :N_CAT].astype(jnp.int32) + offsets).reshape(R * N_CAT)
    cont = xf[:, N_CAT:]
    tab_flat = tables.reshape(N_CAT * CAT_CARD, EMB_DIM)
    embs = _sc_gather(tab_flat, gidx).reshape(R, N_CAT * EMB_DIM)
    W1 = W[: N_CAT * EMB_DIM]
    W2 = W[N_CAT * EMB_DIM :]
    out = _tc_matmul(embs, cont, W1, W2, b.reshape(1, HIDDEN))
    return out.reshape(bs, ms, nr, HIDDEN)


# speedup vs baseline: 36.0058x; 4.5953x over previous
import functools

import jax
import jax.numpy as jnp
from jax import lax
from jax.experimental import pallas as pl
from jax.experimental.pallas import tpu as pltpu
from jax.experimental.pallas import tpu_sc as plsc

N_CAT = 26
CAT_CARD = 100000
EMB_DIM = 16
HIDDEN = 128
F = N_CAT * EMB_DIM

NUM_CORES = 2
NUM_SUBCORES = 16
NUM_WORKERS = NUM_CORES * NUM_SUBCORES
ROWS_PER_W = F // NUM_WORKERS


def _sc_gather_t(t3, idx3, n_rb):
    mesh = plsc.VectorSubcoreMesh(core_axis_name="c", subcore_axis_name="s")
    HALF = n_rb // 2

    @functools.partial(
        pl.kernel,
        out_type=jax.ShapeDtypeStruct((F, n_rb, 128), jnp.float32),
        mesh=mesh,
        compiler_params=pltpu.CompilerParams(
            use_tc_tiling_on_sc=True, needs_layout_passes=False
        ),
        scratch_types=[
            pltpu.VMEM((CAT_CARD,), jnp.float32),
            pltpu.VMEM((HALF, 128), jnp.int32),
            pltpu.VMEM((HALF, 128), jnp.float32),
        ],
    )
    def k(t_hbm, idx_hbm, out_hbm, row_v, idx_v, out_v):
        w = lax.axis_index("s") * NUM_CORES + lax.axis_index("c")
        for j in range(ROWS_PER_W):
            f = w * ROWS_PER_W + j
            col = f // EMB_DIM
            e = f % EMB_DIM
            pltpu.sync_copy(t_hbm.at[col, e], row_v)
            for h in range(2):
                pltpu.sync_copy(idx_hbm.at[col, pl.ds(h * HALF, HALF)], idx_v)

                def body(kk, _):
                    rr = kk // 8
                    cc = (kk % 8) * 16
                    vidx = idx_v[rr, pl.ds(cc, 16)]
                    out_v[rr, pl.ds(cc, 16)] = plsc.load_gather(row_v, [vidx])
                    return 0

                lax.fori_loop(0, HALF * 8, body, 0)
                pltpu.sync_copy(out_v, out_hbm.at[f, pl.ds(h * HALF, HALF)])

    return k(t3, idx3)


def _tc_matmul_t(embsT3, cont, W1, W2, b2):
    n_rb = embsT3.shape[1]
    RB = 8
    BR = RB * 128
    C = cont.shape[1]

    def mm(e_ref, c_ref, w1_ref, w2_ref, b_ref, o_ref):
        e = e_ref[...].reshape(F, BR)
        acc = lax.dot_general(
            e, w1_ref[...], (((0,), (0,)), ((), ())),
            preferred_element_type=jnp.float32,
        )
        acc = acc + jnp.dot(c_ref[...], w2_ref[...],
                            preferred_element_type=jnp.float32)
        o_ref[...] = acc + b_ref[...]

    return pl.pallas_call(
        mm,
        grid=(n_rb // RB,),
        in_specs=[
            pl.BlockSpec((F, RB, 128), lambda i: (0, i, 0)),
            pl.BlockSpec((BR, C), lambda i: (i, 0)),
            pl.BlockSpec((F, HIDDEN), lambda i: (0, 0)),
            pl.BlockSpec((C, HIDDEN), lambda i: (0, 0)),
            pl.BlockSpec((1, HIDDEN), lambda i: (0, 0)),
        ],
        out_specs=pl.BlockSpec((BR, HIDDEN), lambda i: (i, 0)),
        out_shape=jax.ShapeDtypeStruct((n_rb * 128, HIDDEN), jnp.float32),
    )(embsT3, cont, W1, W2, b2)


def kernel(x, tables, W, b):
    bs, ms, nr, d = x.shape
    R = bs * ms * nr
    n_rb = R // 128
    xf = x.reshape(R, d)
    t3 = tables.transpose(0, 2, 1)
    idx3 = xf[:, :N_CAT].astype(jnp.int32).T.reshape(N_CAT, n_rb, 128)
    cont = xf[:, N_CAT:]
    embsT3 = _sc_gather_t(t3, idx3, n_rb)
    W1 = W[:F]
    W2 = W[F:]
    out = _tc_matmul_t(embsT3, cont, W1, W2, b.reshape(1, HIDDEN))
    return out.reshape(bs, ms, nr, HIDDEN)


# speedup vs baseline: 46.6479x; 1.2956x over previous
import functools

import jax
import jax.numpy as jnp
from jax import lax
from jax.experimental import pallas as pl
from jax.experimental.pallas import tpu as pltpu
from jax.experimental.pallas import tpu_sc as plsc

N_CAT = 26
CAT_CARD = 100000
EMB_DIM = 16
HIDDEN = 128
F = N_CAT * EMB_DIM

NUM_CORES = 2
NUM_SUBCORES = 16
NUM_WORKERS = NUM_CORES * NUM_SUBCORES
ROWS_PER_W = F // NUM_WORKERS


def _sc_gather_t(t3, idx3, n_rb):
    mesh = plsc.VectorSubcoreMesh(core_axis_name="c", subcore_axis_name="s")
    NQ = 4
    Q = n_rb // NQ

    @functools.partial(
        pl.kernel,
        out_type=jax.ShapeDtypeStruct((F, n_rb, 128), jnp.float32),
        mesh=mesh,
        compiler_params=pltpu.CompilerParams(
            use_tc_tiling_on_sc=True, needs_layout_passes=False
        ),
        scratch_types=[
            pltpu.VMEM((CAT_CARD,), jnp.float32),
            pltpu.VMEM((2, Q, 128), jnp.int32),
            pltpu.VMEM((2, Q, 128), jnp.float32),
            pltpu.SemaphoreType.DMA,
            pltpu.SemaphoreType.DMA,
            pltpu.SemaphoreType.DMA,
        ],
    )
    def k(t_hbm, idx_hbm, out_hbm, row_v, idx_v, out_v, rsem, isem, osem):
        w = lax.axis_index("s") * NUM_CORES + lax.axis_index("c")

        def row_copy(j):
            f = w * ROWS_PER_W + j
            return pltpu.async_copy(
                t_hbm.at[f // EMB_DIM, f % EMB_DIM], row_v, rsem
            )

        def idx_copy(j, q, s):
            col = (w * ROWS_PER_W + j) // EMB_DIM
            return pltpu.async_copy(
                idx_hbm.at[col, pl.ds(q * Q, Q)], idx_v.at[s], isem
            )

        rcp = row_copy(0)
        icp = idx_copy(0, 0, 0)
        ocp = [None, None]
        for j in range(ROWS_PER_W):
            f = w * ROWS_PER_W + j
            rcp.wait()
            for q in range(NQ):
                s = q & 1
                icp.wait()
                if q < NQ - 1:
                    icp = idx_copy(j, q + 1, 1 - s)
                elif j < ROWS_PER_W - 1:
                    icp = idx_copy(j + 1, 0, 1 - s)
                if ocp[s] is not None:
                    ocp[s].wait()

                def body(rr, _, s=s):
                    for u in range(8):
                        vidx = idx_v[s, rr, pl.ds(u * 16, 16)]
                        out_v[s, rr, pl.ds(u * 16, 16)] = plsc.load_gather(
                            row_v, [vidx]
                        )
                    return 0

                lax.fori_loop(0, Q, body, 0)
                if q == NQ - 1 and j < ROWS_PER_W - 1:
                    rcp = row_copy(j + 1)
                ocp[s] = pltpu.async_copy(
                    out_v.at[s], out_hbm.at[f, pl.ds(q * Q, Q)], osem
                )
        for cp in ocp:
            if cp is not None:
                cp.wait()

    return k(t3, idx3)


def _tc_matmul_t(embsT3, cont, W1, W2, b2):
    n_rb = embsT3.shape[1]
    RB = 8
    BR = RB * 128
    C = cont.shape[1]

    def mm(e_ref, c_ref, w1_ref, w2_ref, b_ref, o_ref):
        e = e_ref[...].reshape(F, BR)
        acc = lax.dot_general(
            e, w1_ref[...], (((0,), (0,)), ((), ())),
            preferred_element_type=jnp.float32,
        )
        acc = acc + jnp.dot(c_ref[...], w2_ref[...],
                            preferred_element_type=jnp.float32)
        o_ref[...] = acc + b_ref[...]

    return pl.pallas_call(
        mm,
        grid=(n_rb // RB,),
        in_specs=[
            pl.BlockSpec((F, RB, 128), lambda i: (0, i, 0)),
            pl.BlockSpec((BR, C), lambda i: (i, 0)),
            pl.BlockSpec((F, HIDDEN), lambda i: (0, 0)),
            pl.BlockSpec((C, HIDDEN), lambda i: (0, 0)),
            pl.BlockSpec((1, HIDDEN), lambda i: (0, 0)),
        ],
        out_specs=pl.BlockSpec((BR, HIDDEN), lambda i: (i, 0)),
        out_shape=jax.ShapeDtypeStruct((n_rb * 128, HIDDEN), jnp.float32),
    )(embsT3, cont, W1, W2, b2)


def kernel(x, tables, W, b):
    bs, ms, nr, d = x.shape
    R = bs * ms * nr
    n_rb = R // 128
    xf = x.reshape(R, d)
    t3 = tables.transpose(0, 2, 1)
    idx3 = xf[:, :N_CAT].astype(jnp.int32).T.reshape(N_CAT, n_rb, 128)
    cont = xf[:, N_CAT:]
    embsT3 = _sc_gather_t(t3, idx3, n_rb)
    W1 = W[:F]
    W2 = W[F:]
    out = _tc_matmul_t(embsT3, cont, W1, W2, b.reshape(1, HIDDEN))
    return out.reshape(bs, ms, nr, HIDDEN)


# speedup vs baseline: 50.7486x; 1.0879x over previous
import functools

import jax
import jax.numpy as jnp
from jax import lax
from jax.experimental import pallas as pl
from jax.experimental.pallas import tpu as pltpu
from jax.experimental.pallas import tpu_sc as plsc

N_CAT = 26
CAT_CARD = 100000
EMB_DIM = 16
HIDDEN = 128
F = N_CAT * EMB_DIM

NUM_CORES = 2
NUM_SUBCORES = 16
NUM_WORKERS = NUM_CORES * NUM_SUBCORES
ROWS_PER_W = F // NUM_WORKERS


def _sc_gather_t(t3, idx3, n_rb):
    mesh = plsc.VectorSubcoreMesh(core_axis_name="c", subcore_axis_name="s")
    NQ = 4
    Q = n_rb // NQ

    @functools.partial(
        pl.kernel,
        out_type=jax.ShapeDtypeStruct((F, n_rb, 128), jnp.float32),
        mesh=mesh,
        compiler_params=pltpu.CompilerParams(
            use_tc_tiling_on_sc=True, needs_layout_passes=False
        ),
        scratch_types=[
            pltpu.VMEM((CAT_CARD,), jnp.float32),
            pltpu.VMEM((2, Q, 128), jnp.int32),
            pltpu.VMEM((2, Q, 128), jnp.float32),
            pltpu.SemaphoreType.DMA,
            pltpu.SemaphoreType.DMA,
            pltpu.SemaphoreType.DMA,
        ],
    )
    def k(t_hbm, idx_hbm, out_hbm, row_v, idx_v, out_v, rsem, isem, osem):
        w = lax.axis_index("s") * NUM_CORES + lax.axis_index("c")

        def row_copy(j):
            f = w * ROWS_PER_W + j
            return pltpu.async_copy(
                t_hbm.at[f // EMB_DIM, f % EMB_DIM], row_v, rsem
            )

        def idx_copy(j, q, s):
            col = (w * ROWS_PER_W + j) // EMB_DIM
            return pltpu.async_copy(
                idx_hbm.at[col, pl.ds(q * Q, Q)], idx_v.at[s], isem
            )

        rcp = row_copy(0)
        icp = idx_copy(0, 0, 0)
        ocp = [None, None]
        for j in range(ROWS_PER_W):
            f = w * ROWS_PER_W + j
            rcp.wait()
            for q in range(NQ):
                s = q & 1
                icp.wait()
                if q < NQ - 1:
                    icp = idx_copy(j, q + 1, 1 - s)
                elif j < ROWS_PER_W - 1:
                    icp = idx_copy(j + 1, 0, 1 - s)
                if ocp[s] is not None:
                    ocp[s].wait()

                def body(rr, _, s=s):
                    for u in range(8):
                        vidx = idx_v[s, rr, pl.ds(u * 16, 16)]
                        out_v[s, rr, pl.ds(u * 16, 16)] = plsc.load_gather(
                            row_v, [vidx]
                        )
                    return 0

                lax.fori_loop(0, Q, body, 0)
                if q == NQ - 1 and j < ROWS_PER_W - 1:
                    rcp = row_copy(j + 1)
                ocp[s] = pltpu.async_copy(
                    out_v.at[s], out_hbm.at[f, pl.ds(q * Q, Q)], osem
                )
        for cp in ocp:
            if cp is not None:
                cp.wait()

    return k(t3, idx3)


def _tc_matmul_t(embsT3, cont, W1, W2, b2, bs, ms, nr):
    n_rb = embsT3.shape[1]
    BR = nr * bs
    RB = BR // 128
    C = cont.shape[1]

    def mm(e_ref, c_ref, w1_ref, w2_ref, b_ref, o_ref):
        e = e_ref[...].reshape(F, BR)
        acc = lax.dot_general(
            e, w1_ref[...], (((0,), (0,)), ((), ())),
            preferred_element_type=jnp.float32,
        )
        acc = acc + jnp.dot(c_ref[...], w2_ref[...],
                            preferred_element_type=jnp.float32)
        acc = acc + b_ref[...]
        o_ref[...] = acc.reshape(nr, bs, HIDDEN).transpose(1, 0, 2).reshape(
            bs, 1, nr, HIDDEN
        )

    return pl.pallas_call(
        mm,
        grid=(ms,),
        in_specs=[
            pl.BlockSpec((F, RB, 128), lambda i: (0, i, 0)),
            pl.BlockSpec((BR, C), lambda i: (i, 0)),
            pl.BlockSpec((F, HIDDEN), lambda i: (0, 0)),
            pl.BlockSpec((C, HIDDEN), lambda i: (0, 0)),
            pl.BlockSpec((1, HIDDEN), lambda i: (0, 0)),
        ],
        out_specs=pl.BlockSpec((bs, 1, nr, HIDDEN), lambda i: (0, i, 0, 0)),
        out_shape=jax.ShapeDtypeStruct((bs, ms, nr, HIDDEN), jnp.float32),
    )(embsT3, cont, W1, W2, b2)


def kernel(x, tables, W, b):
    bs, ms, nr, d = x.shape
    R = bs * ms * nr
    n_rb = R // 128
    t3 = tables.transpose(0, 2, 1)
    xt = x.transpose(1, 2, 3, 0)
    idx3 = (
        xt[:, :, :N_CAT, :]
        .astype(jnp.int32)
        .transpose(2, 0, 1, 3)
        .reshape(N_CAT, n_rb, 128)
    )
    cont = xt[:, :, N_CAT:, :].transpose(0, 1, 3, 2).reshape(R, d - N_CAT)
    embsT3 = _sc_gather_t(t3, idx3, n_rb)
    W1 = W[:F]
    W2 = W[F:]
    return _tc_matmul_t(embsT3, cont, W1, W2, b.reshape(1, HIDDEN), bs, ms, nr)


# speedup vs baseline: 56.8661x; 1.1205x over previous
import functools

import jax
import jax.numpy as jnp
from jax import lax
from jax.experimental import pallas as pl
from jax.experimental.pallas import tpu as pltpu
from jax.experimental.pallas import tpu_sc as plsc

N_CAT = 26
CAT_CARD = 100000
EMB_DIM = 16
HIDDEN = 128
F = N_CAT * EMB_DIM

NUM_CORES = 2
NUM_SUBCORES = 16
NUM_WORKERS = NUM_CORES * NUM_SUBCORES
ROWS_PER_W = F // NUM_WORKERS


def _sc_gather_t(t3, idx3, n_rb):
    mesh = plsc.VectorSubcoreMesh(core_axis_name="c", subcore_axis_name="s")
    NQ = 4
    Q = n_rb // NQ

    @functools.partial(
        pl.kernel,
        out_type=jax.ShapeDtypeStruct((F, n_rb, 128), jnp.float32),
        mesh=mesh,
        compiler_params=pltpu.CompilerParams(
            use_tc_tiling_on_sc=True, needs_layout_passes=False
        ),
        scratch_types=[
            pltpu.VMEM((CAT_CARD,), jnp.float32),
            pltpu.VMEM((2, Q, 128), jnp.int32),
            pltpu.VMEM((2, Q, 128), jnp.float32),
            pltpu.SemaphoreType.DMA,
            pltpu.SemaphoreType.DMA,
            pltpu.SemaphoreType.DMA,
        ],
    )
    def k(t_hbm, idx_hbm, out_hbm, row_v, idx_v, out_v, rsem, isem, osem):
        w = lax.axis_index("s") * NUM_CORES + lax.axis_index("c")

        def row_copy(j):
            f = w * ROWS_PER_W + j
            return pltpu.async_copy(
                t_hbm.at[f // EMB_DIM, f % EMB_DIM], row_v, rsem
            )

        def idx_copy(j, q, s):
            col = (w * ROWS_PER_W + j) // EMB_DIM
            return pltpu.async_copy(
                idx_hbm.at[col, pl.ds(q * Q, Q)], idx_v.at[s], isem
            )

        rcp = row_copy(0)
        icp = idx_copy(0, 0, 0)
        ocp = [None, None]
        for j in range(ROWS_PER_W):
            f = w * ROWS_PER_W + j
            rcp.wait()
            for q in range(NQ):
                s = q & 1
                icp.wait()
                if q < NQ - 1:
                    icp = idx_copy(j, q + 1, 1 - s)
                elif j < ROWS_PER_W - 1:
                    icp = idx_copy(j + 1, 0, 1 - s)
                if ocp[s] is not None:
                    ocp[s].wait()

                @plsc.parallel_loop(0, Q, 1, unroll=1)
                def _(rr, s=s):
                    for u in range(8):
                        vidx = idx_v[s, rr, pl.ds(u * 16, 16)]
                        out_v[s, rr, pl.ds(u * 16, 16)] = plsc.load_gather(
                            row_v, [vidx]
                        )
                if q == NQ - 1 and j < ROWS_PER_W - 1:
                    rcp = row_copy(j + 1)
                ocp[s] = pltpu.async_copy(
                    out_v.at[s], out_hbm.at[f, pl.ds(q * Q, Q)], osem
                )
        for cp in ocp:
            if cp is not None:
                cp.wait()

    return k(t3, idx3)


def _tc_matmul_t(embsT3, cont, W1, W2, b2, bs, ms, nr):
    n_rb = embsT3.shape[1]
    BR = nr * bs
    RB = BR // 128
    C = cont.shape[1]

    def mm(e_ref, c_ref, w1_ref, w2_ref, b_ref, o_ref):
        e = e_ref[...].reshape(F, BR)
        acc = lax.dot_general(
            e, w1_ref[...], (((0,), (0,)), ((), ())),
            preferred_element_type=jnp.float32,
        )
        acc = acc + jnp.dot(c_ref[...], w2_ref[...],
                            preferred_element_type=jnp.float32)
        acc = acc + b_ref[...]
        o_ref[...] = acc.reshape(nr, bs, HIDDEN).transpose(1, 0, 2).reshape(
            bs, 1, nr, HIDDEN
        )

    return pl.pallas_call(
        mm,
        grid=(ms,),
        in_specs=[
            pl.BlockSpec((F, RB, 128), lambda i: (0, i, 0)),
            pl.BlockSpec((BR, C), lambda i: (i, 0)),
            pl.BlockSpec((F, HIDDEN), lambda i: (0, 0)),
            pl.BlockSpec((C, HIDDEN), lambda i: (0, 0)),
            pl.BlockSpec((1, HIDDEN), lambda i: (0, 0)),
        ],
        out_specs=pl.BlockSpec((bs, 1, nr, HIDDEN), lambda i: (0, i, 0, 0)),
        out_shape=jax.ShapeDtypeStruct((bs, ms, nr, HIDDEN), jnp.float32),
    )(embsT3, cont, W1, W2, b2)


def kernel(x, tables, W, b):
    bs, ms, nr, d = x.shape
    R = bs * ms * nr
    n_rb = R // 128
    t3 = tables.transpose(0, 2, 1)
    xt = x.transpose(1, 2, 3, 0)
    idx3 = (
        xt[:, :, :N_CAT, :]
        .astype(jnp.int32)
        .transpose(2, 0, 1, 3)
        .reshape(N_CAT, n_rb, 128)
    )
    cont = xt[:, :, N_CAT:, :].transpose(0, 1, 3, 2).reshape(R, d - N_CAT)
    embsT3 = _sc_gather_t(t3, idx3, n_rb)
    W1 = W[:F]
    W2 = W[F:]
    return _tc_matmul_t(embsT3, cont, W1, W2, b.reshape(1, HIDDEN), bs, ms, nr)
